# baseline (speedup 1.0000x reference)
import functools
import itertools

import jax
import jax.numpy as jnp
from jax import lax
from jax.experimental import pallas as pl
from jax.experimental.pallas import tpu as pltpu
from jax.experimental.pallas import tpu_sc as plsc

DIM = 3
KS = 4
K = KS ** DIM
KT = K + 1
F = 128
N = 10000
NC = 2
NS = 16
NW = NC * NS
CH = 16
S = 8
GPS = 16
SCB = GPS * CH
NDEG = 10240
NACC = 10112
BN = 400


NPAIR = 48
BN2 = 200


def _xw_body(x_ref, w_ref, o_ref):
    xb = x_ref[...]
    for j in range(NPAIR):
        o_ref[j] = jnp.dot(xb, w_ref[j], preferred_element_type=jnp.float32)


def _compute_xw_pairs(x, weight):
    wp = weight[:K].reshape(16, KS, F, F)
    pairs = [jnp.concatenate([wp[:, pq], wp[:, pq + 1]], axis=-1)
             for pq in range(3)]
    w2 = jnp.stack(pairs, axis=1).reshape(NPAIR, F, 2 * F)
    return pl.pallas_call(
        _xw_body,
        grid=(N // BN2,),
        in_specs=[
            pl.BlockSpec((BN2, F), lambda nb: (nb, 0)),
            pl.BlockSpec((NPAIR, F, 2 * F), lambda nb: (0, 0, 0)),
        ],
        out_specs=pl.BlockSpec((NPAIR, BN2, 2 * F), lambda nb: (0, nb, 0)),
        out_shape=jax.ShapeDtypeStruct((NPAIR, N, 2 * F), jnp.float32),
    )(x.astype(jnp.bfloat16), w2.astype(jnp.bfloat16))


def _root_body(x_ref, w_ref, o_ref):
    o_ref[...] = jnp.dot(x_ref[...], w_ref[...],
                         preferred_element_type=jnp.float32)


def _compute_root(x, w_root):
    return pl.pallas_call(
        _root_body,
        grid=(N // BN,),
        in_specs=[
            pl.BlockSpec((BN, F), lambda nb: (nb, 0)),
            pl.BlockSpec((F, F), lambda nb: (0, 0)),
        ],
        out_specs=pl.BlockSpec((BN, F), lambda nb: (nb, 0)),
        out_shape=jax.ShapeDtypeStruct((N, F), jnp.float32),
    )(x.astype(jnp.bfloat16), w_root.astype(jnp.bfloat16))


def _splat(vec, lane):
    idx = jnp.full((16, 1), lane, jnp.int32)
    dnums = lax.GatherDimensionNumbers(
        offset_dims=(), collapsed_slice_dims=(0,), start_index_map=(0,))
    return lax.gather(vec, idx, dnums, (1,),
                      mode=lax.GatherScatterMode.PROMISE_IN_BOUNDS)


def _make_sc_kernel(E, ept_pad, nsc):
    mesh = plsc.VectorSubcoreMesh(core_axis_name="c", subcore_axis_name="s")
    rows_per_sub = NACC // NS
    chunks = ((0, 128), (128, 128), (256, 128), (384, 128), (512, 120))
    deg_per_sub = NDEG // NS

    @functools.partial(
        pl.kernel,
        out_type=(
            jax.ShapeDtypeStruct((NC, NACC, F), jnp.float32),
            jax.ShapeDtypeStruct((NC * NDEG,), jnp.float32),
        ),
        mesh=mesh,
        scratch_types=[
            pltpu.VMEM_SHARED((NACC, F), jnp.float32),
            pltpu.VMEM_SHARED((NDEG,), jnp.float32),
            pltpu.VMEM((2 * SCB,), jnp.float32),
            pltpu.VMEM((2 * SCB,), jnp.float32),
            pltpu.VMEM((2 * SCB,), jnp.float32),
            pltpu.VMEM((2, GPS, CH), jnp.int32),
            pltpu.VMEM((2 * SCB,), jnp.int32),
            pltpu.VMEM((2 * SCB * 4,), jnp.int32),
            pltpu.VMEM((2, DIM, SCB), jnp.float32),
            pltpu.VMEM((2, CH * 4, 2, F), jnp.float32),
            pltpu.VMEM((2, CH, F), jnp.float32),
            pltpu.VMEM((128,), jnp.float32),
            pltpu.VMEM((128,), jnp.float32),
            pltpu.SemaphoreType.DMA,
            pltpu.SemaphoreType.DMA,
            pltpu.SemaphoreType.DMA,
            pltpu.SemaphoreType.DMA,
            pltpu.SemaphoreType.DMA,
            pltpu.SemaphoreType.DMA,
        ],
    )
    def sc_kernel(p0_hbm, p1_hbm, p2_hbm, row_hbm, col_hbm, xw_hbm,
                  acc_out, deg_out,
                  accum_sh, deg_sh, p0v, p1v, p2v, rowv, colv, idxv,
                  frv, rows2, eout2, dz, ones,
                  isem, gsem0, gsem1, ssem0, ssem1, dsem):
        cid = lax.axis_index("c")
        sid = lax.axis_index("s")
        wid = sid * NC + cid
        gsems = (gsem0, gsem1)
        ssems = (ssem0, ssem1)

        def _zloop(i, _):
            for j in range(F // 16):
                eout2[0, i, pl.ds(16 * j, 16)] = jnp.zeros((16,), jnp.float32)
            return 0
        lax.fori_loop(0, CH, _zloop, 0)
        for j in range(128 // 16):
            dz[pl.ds(16 * j, 16)] = jnp.zeros((16,), jnp.float32)
            ones[pl.ds(16 * j, 16)] = jnp.ones((16,), jnp.float32)

        for k in range(rows_per_sub // CH):
            pltpu.sync_copy(
                eout2.at[0],
                accum_sh.at[pl.ds(sid * rows_per_sub + k * CH, CH)])
        pltpu.sync_copy(
            eout2.at[0, pl.ds(0, rows_per_sub % CH)],
            accum_sh.at[pl.ds(sid * rows_per_sub
                              + (rows_per_sub // CH) * CH,
                              rows_per_sub % CH)])
        for k in range(deg_per_sub // 128):
            pltpu.sync_copy(dz, deg_sh.at[pl.ds(sid * deg_per_sub + k * 128,
                                                128)])
        plsc.subcore_barrier()

        base0 = wid * ept_pad
        rbase0 = (wid * ept_pad) // CH
        lanes = lax.iota(jnp.int32, 16)
        inps = ((p0_hbm, p0v), (p1_hbm, p1v), (p2_hbm, p2v),
                (col_hbm, colv))

        def _issue_inputs(i):
            off = base0 + i * SCB
            nb = i % 2
            for hbm, buf in inps:
                pltpu.async_copy(hbm.at[pl.ds(off, SCB)],
                                 buf.at[pl.ds(nb * SCB, SCB)], isem)
            pltpu.async_copy(
                row_hbm.at[pl.ds(pl.multiple_of(rbase0 + i * GPS, 8), GPS), :],
                rowv.at[nb], isem)

        def _wait_inputs(i):
            off = base0 + i * SCB
            nb = i % 2
            for hbm, buf in inps:
                pltpu.make_async_copy(
                    hbm.at[pl.ds(off, SCB)],
                    buf.at[pl.ds(nb * SCB, SCB)], isem).wait()
            pltpu.make_async_copy(
                row_hbm.at[pl.ds(pl.multiple_of(rbase0 + i * GPS, 8), GPS), :],
                rowv.at[nb], isem).wait()

        def _basis(i, nb, g):
            e0 = g * CH
            colx = colv[pl.ds(nb * SCB + e0, CH)]
            lo = []
            for d, pv in enumerate((p0v, p1v, p2v)):
                v = pv[pl.ds(nb * SCB + e0, CH)] * float(KS - 1)
                li = jnp.minimum(v.astype(jnp.int32), KS - 2)
                lo.append(li)
                frv[nb, d, pl.ds(e0, CH)] = v - li.astype(jnp.float32)
            for q, (b0, b1) in enumerate(itertools.product((0, 1), repeat=2)):
                pairidx = ((lo[0] + b0) * KS + (lo[1] + b1)) * 3 + lo[2]
                idxv[pl.ds(nb * (SCB * 4) + g * (CH * 4) + q * CH,
                           CH)] = pairidx * N + colx

        def _gather_desc(nb, g, p):
            return pltpu.make_async_copy(
                xw_hbm.at[idxv.at[pl.ds(nb * (SCB * 4) + g * (CH * 4), CH * 4)]],
                rows2.at[p], gsems[p])

        def _scat_desc(b, g, p):
            return pltpu.make_async_copy(
                eout2.at[p], accum_sh.at[rowv.at[b, g]], ssems[p])

        def _compute(b, g, p):
            fv = [frv[b, d, pl.ds(g * CH, CH)] for d in range(DIM)]

            def _edge(eh, _):
                for e2 in range(1):
                    e = eh + e2
                    f0 = _splat(fv[0], e)
                    f1 = _splat(fv[1], e)
                    f2 = _splat(fv[2], e)
                    g0, g1, g2 = 1.0 - f0, 1.0 - f1, 1.0 - f2
                    t00, t01 = g0 * g1, g0 * f1
                    t10, t11 = f0 * g1, f0 * f1
                    sp = [t00 * g2, t00 * f2, t01 * g2, t01 * f2,
                          t10 * g2, t10 * f2, t11 * g2, t11 * f2]
                    for f8 in range(F // 16):
                        acc = sp[0] * rows2[p, 0 * CH + e, 0,
                                            pl.ds(f8 * 16, 16)]
                        for si in range(1, S):
                            acc = acc + sp[si] * rows2[p, (si // 2) * CH + e,
                                                       si % 2,
                                                       pl.ds(f8 * 16, 16)]
                        eout2[p, e, pl.ds(f8 * 16, 16)] = acc
                return 0
            lax.fori_loop(0, CH, _edge, 0)

        def _group(i, b, gp, g, p):
            if p == 0:
                _gather_desc(b, g + 1, 1).start()
            else:
                @pl.when(gp < GPS // 2 - 1)
                def _():
                    _gather_desc(b, g + 1, 0).start()

                @pl.when(jnp.logical_and(gp == GPS // 2 - 1, i < nsc - 1))
                def _():
                    _gather_desc(1 - b, 0, 0).start()
            @pl.when(gp >= 1)
            def _():
                _scat_desc(b, g - 2, p).wait()
            _gather_desc(b, g, p).wait()
            _compute(b, g, p)
            pltpu.async_copy(eout2.at[p], accum_sh.at[rowv.at[b, g]],
                             ssems[p], add=True)
            _basis(i + 1, 1 - b, g)

        _issue_inputs(0)
        _wait_inputs(0)

        def _basis0(g, _):
            _basis(0, 0, g)
            return 0
        lax.fori_loop(0, GPS, _basis0, 0)
        if nsc > 1:
            _issue_inputs(1)
        _gather_desc(0, 0, 0).start()

        def _superchunk(i, _):
            b = i % 2
            @pl.when(i + 1 < nsc)
            def _():
                _wait_inputs(i + 1)

            for gg in range(GPS):
                pltpu.async_copy(ones.at[pl.ds(0, CH)],
                                 deg_sh.at[rowv.at[b, gg]], dsem, add=True)

            def _pair(gp, _2):
                _group(i, b, gp, 2 * gp, 0)
                _group(i, b, gp, 2 * gp + 1, 1)
                return 0
            lax.fori_loop(0, GPS // 2, _pair, 0)

            _scat_desc(b, GPS - 2, 0).wait()
            _scat_desc(b, GPS - 1, 1).wait()
            for gg in range(GPS):
                pltpu.make_async_copy(ones.at[pl.ds(0, CH)],
                                      deg_sh.at[rowv.at[b, gg]], dsem).wait()

            @pl.when(i + 2 < nsc)
            def _():
                _issue_inputs(i + 2)
            return 0

        lax.fori_loop(0, nsc, _superchunk, 0)
        plsc.subcore_barrier()

        for off0, cl in chunks:
            off = sid * rows_per_sub + off0
            pltpu.sync_copy(accum_sh.at[pl.ds(off, cl)],
                            acc_out.at[cid, pl.ds(off, cl)])
        pltpu.sync_copy(
            deg_sh.at[pl.ds(sid * deg_per_sub, deg_per_sub)],
            deg_out.at[pl.ds(cid * NDEG + sid * deg_per_sub, deg_per_sub)])

    return sc_kernel


def _combine_body(a_ref, d_ref, r_ref, b_ref, o_ref):
    a = a_ref[0] + a_ref[1]
    d = d_ref[0] + d_ref[1]
    o_ref[...] = a / jnp.maximum(d, 1.0) + r_ref[...] + b_ref[...]


def _combine(acc, deg, root, bias):
    return pl.pallas_call(
        _combine_body,
        grid=(N // BN,),
        in_specs=[
            pl.BlockSpec((NC, BN, F), lambda i: (0, i, 0)),
            pl.BlockSpec((NC, BN, 1), lambda i: (0, i, 0)),
            pl.BlockSpec((BN, F), lambda i: (i, 0)),
            pl.BlockSpec((1, F), lambda i: (0, 0)),
        ],
        out_specs=pl.BlockSpec((BN, F), lambda i: (i, 0)),
        out_shape=jax.ShapeDtypeStruct((N, F), jnp.float32),
    )(acc, deg, root, bias)


def kernel(x, edge_index, pseudo, weight, bias):
    E = edge_index.shape[1]
    ept = -(-E // NW)
    nsc = -(-ept // SCB)
    ept_pad = nsc * SCB
    e_pad = ept_pad * NW

    xwp = _compute_xw_pairs(x, weight)
    xw_flat = xwp.reshape(NPAIR * N, 2, F)
    root = _compute_root(x, weight[K])

    row = edge_index[0]
    col = edge_index[1]
    pad = e_pad - E
    rowp = jnp.pad(row, (0, pad),
                   constant_values=NACC - 1).reshape(e_pad // CH, CH)
    colp = jnp.pad(col, (0, pad))
    pp = [jnp.pad(pseudo[:, d], (0, pad)) for d in range(DIM)]

    sc = _make_sc_kernel(E, ept_pad, nsc)
    acc, deg = sc(pp[0], pp[1], pp[2], rowp, colp, xw_flat)

    deg3 = deg.reshape(NC, NDEG)[:, :N].reshape(NC, N, 1)
    return _combine(acc, deg3, root, bias.reshape(1, F))


# speedup vs baseline: 1.5797x; 1.5797x over previous
import functools
import itertools

import jax
import jax.numpy as jnp
from jax import lax
from jax.experimental import pallas as pl
from jax.experimental.pallas import tpu as pltpu
from jax.experimental.pallas import tpu_sc as plsc

DIM = 3
KS = 4
K = KS ** DIM
KT = K + 1
F = 128
N = 10000
NC = 2
NS = 16
NW = NC * NS
CH = 16
S = 8
GPS = 16
SCB = GPS * CH
NDEG = 10240
NACC = 10112
BN = 400


def _xw_body(x_ref, w_ref, o_ref):
    xb = x_ref[...]
    for k in range(KT):
        o_ref[k] = jnp.dot(xb, w_ref[k], preferred_element_type=jnp.float32)


def _compute_xw(x, weight):
    return pl.pallas_call(
        _xw_body,
        grid=(N // BN,),
        in_specs=[
            pl.BlockSpec((BN, F), lambda nb: (nb, 0)),
            pl.BlockSpec((KT, F, F), lambda nb: (0, 0, 0)),
        ],
        out_specs=pl.BlockSpec((KT, BN, F), lambda nb: (0, nb, 0)),
        out_shape=jax.ShapeDtypeStruct((KT, N, F), jnp.float32),
    )(x.astype(jnp.bfloat16), weight.astype(jnp.bfloat16))


def _splat(vec, lane):
    idx = jnp.full((16, 1), lane, jnp.int32)
    dnums = lax.GatherDimensionNumbers(
        offset_dims=(), collapsed_slice_dims=(0,), start_index_map=(0,))
    return lax.gather(vec, idx, dnums, (1,),
                      mode=lax.GatherScatterMode.PROMISE_IN_BOUNDS)


def _make_sc_kernel(E, ept_pad, nsc):
    mesh = plsc.VectorSubcoreMesh(core_axis_name="c", subcore_axis_name="s")
    rows_per_sub = NACC // NS
    chunks = ((0, 128), (128, 128), (256, 128), (384, 128), (512, 120))
    deg_per_sub = NDEG // NS

    @functools.partial(
        pl.kernel,
        out_type=(
            jax.ShapeDtypeStruct((NC, NACC, F), jnp.float32),
            jax.ShapeDtypeStruct((NC * NDEG,), jnp.float32),
        ),
        mesh=mesh,
        scratch_types=[
            pltpu.VMEM_SHARED((NACC, F), jnp.float32),
            pltpu.VMEM_SHARED((NDEG,), jnp.float32),
            pltpu.VMEM((2 * SCB,), jnp.float32),
            pltpu.VMEM((2 * SCB,), jnp.float32),
            pltpu.VMEM((2 * SCB,), jnp.float32),
            pltpu.VMEM((2, GPS, CH), jnp.int32),
            pltpu.VMEM((2 * SCB,), jnp.int32),
            pltpu.VMEM((2 * SCB * S,), jnp.int32),
            pltpu.VMEM((2, DIM, SCB), jnp.float32),
            pltpu.VMEM((2, CH * S, F), jnp.float32),
            pltpu.VMEM((2, CH, F), jnp.float32),
            pltpu.VMEM((128,), jnp.float32),
            pltpu.VMEM((128,), jnp.float32),
            pltpu.SemaphoreType.DMA,
            pltpu.SemaphoreType.DMA,
            pltpu.SemaphoreType.DMA,
            pltpu.SemaphoreType.DMA,
            pltpu.SemaphoreType.DMA,
            pltpu.SemaphoreType.DMA,
        ],
    )
    def sc_kernel(p0_hbm, p1_hbm, p2_hbm, row_hbm, col_hbm, xw_hbm,
                  acc_out, deg_out,
                  accum_sh, deg_sh, p0v, p1v, p2v, rowv, colv, idxv,
                  frv, rows2, eout2, dz, ones,
                  isem, gsem0, gsem1, ssem0, ssem1, dsem):
        cid = lax.axis_index("c")
        sid = lax.axis_index("s")
        wid = sid * NC + cid
        gsems = (gsem0, gsem1)
        ssems = (ssem0, ssem1)

        def _zloop(i, _):
            for j in range(F // 16):
                eout2[0, i, pl.ds(16 * j, 16)] = jnp.zeros((16,), jnp.float32)
            return 0
        lax.fori_loop(0, CH, _zloop, 0)
        for j in range(128 // 16):
            dz[pl.ds(16 * j, 16)] = jnp.zeros((16,), jnp.float32)
            ones[pl.ds(16 * j, 16)] = jnp.ones((16,), jnp.float32)

        for k in range(rows_per_sub // CH):
            pltpu.sync_copy(
                eout2.at[0],
                accum_sh.at[pl.ds(sid * rows_per_sub + k * CH, CH)])
        pltpu.sync_copy(
            eout2.at[0, pl.ds(0, rows_per_sub % CH)],
            accum_sh.at[pl.ds(sid * rows_per_sub
                              + (rows_per_sub // CH) * CH,
                              rows_per_sub % CH)])
        for k in range(deg_per_sub // 128):
            pltpu.sync_copy(dz, deg_sh.at[pl.ds(sid * deg_per_sub + k * 128,
                                                128)])
        plsc.subcore_barrier()

        base0 = wid * ept_pad
        rbase0 = (wid * ept_pad) // CH
        lanes = lax.iota(jnp.int32, 16)
        inps = ((p0_hbm, p0v), (p1_hbm, p1v), (p2_hbm, p2v),
                (col_hbm, colv))

        def _issue_inputs(i):
            off = base0 + i * SCB
            nb = i % 2
            for hbm, buf in inps:
                pltpu.async_copy(hbm.at[pl.ds(off, SCB)],
                                 buf.at[pl.ds(nb * SCB, SCB)], isem)
            pltpu.async_copy(
                row_hbm.at[pl.ds(pl.multiple_of(rbase0 + i * GPS, 8), GPS), :],
                rowv.at[nb], isem)

        def _wait_inputs(i):
            off = base0 + i * SCB
            nb = i % 2
            for hbm, buf in inps:
                pltpu.make_async_copy(
                    hbm.at[pl.ds(off, SCB)],
                    buf.at[pl.ds(nb * SCB, SCB)], isem).wait()
            pltpu.make_async_copy(
                row_hbm.at[pl.ds(pl.multiple_of(rbase0 + i * GPS, 8), GPS), :],
                rowv.at[nb], isem).wait()

        def _basis(i, nb, g):
            e0 = g * CH
            colx = colv[pl.ds(nb * SCB + e0, CH)]
            lo = []
            for d, pv in enumerate((p0v, p1v, p2v)):
                v = pv[pl.ds(nb * SCB + e0, CH)] * float(KS - 1)
                li = jnp.minimum(v.astype(jnp.int32), KS - 2)
                lo.append(li)
                frv[nb, d, pl.ds(e0, CH)] = v - li.astype(jnp.float32)
            for sidx, bits in enumerate(itertools.product((0, 1), repeat=DIM)):
                idxl = jnp.zeros((16,), jnp.int32)
                for d, bit in enumerate(bits):
                    idxl = idxl + (lo[d] + bit) * (KS ** (DIM - 1 - d))
                idxv[pl.ds(nb * (SCB * S) + g * (CH * S) + sidx * CH,
                           CH)] = idxl * N + colx

        def _gather_desc(nb, g, p):
            return pltpu.make_async_copy(
                xw_hbm.at[idxv.at[pl.ds(nb * (SCB * S) + g * (CH * S), CH * S)]],
                rows2.at[p], gsems[p])

        def _scat_desc(b, g, p):
            return pltpu.make_async_copy(
                eout2.at[p], accum_sh.at[rowv.at[b, g]], ssems[p])

        def _compute(b, g, p):
            fv = [frv[b, d, pl.ds(g * CH, CH)] for d in range(DIM)]

            def _edge(eh, _):
                for e2 in range(1):
                    e = eh + e2
                    f0 = _splat(fv[0], e)
                    f1 = _splat(fv[1], e)
                    f2 = _splat(fv[2], e)
                    g0, g1, g2 = 1.0 - f0, 1.0 - f1, 1.0 - f2
                    t00, t01 = g0 * g1, g0 * f1
                    t10, t11 = f0 * g1, f0 * f1
                    sp = [t00 * g2, t00 * f2, t01 * g2, t01 * f2,
                          t10 * g2, t10 * f2, t11 * g2, t11 * f2]
                    for f8 in range(F // 16):
                        acc = sp[0] * rows2[p, 0 * CH + e, pl.ds(f8 * 16, 16)]
                        for si in range(1, S):
                            acc = acc + sp[si] * rows2[p, si * CH + e,
                                                       pl.ds(f8 * 16, 16)]
                        eout2[p, e, pl.ds(f8 * 16, 16)] = acc
                return 0
            lax.fori_loop(0, CH, _edge, 0)

        def _group(i, b, gp, g, p):
            if p == 0:
                _gather_desc(b, g + 1, 1).start()
            else:
                @pl.when(gp < GPS // 2 - 1)
                def _():
                    _gather_desc(b, g + 1, 0).start()

                @pl.when(jnp.logical_and(gp == GPS // 2 - 1, i < nsc - 1))
                def _():
                    _gather_desc(1 - b, 0, 0).start()
            @pl.when(gp >= 1)
            def _():
                _scat_desc(b, g - 2, p).wait()
            _gather_desc(b, g, p).wait()
            _compute(b, g, p)
            pltpu.async_copy(eout2.at[p], accum_sh.at[rowv.at[b, g]],
                             ssems[p], add=True)
            _basis(i + 1, 1 - b, g)

        _issue_inputs(0)
        _wait_inputs(0)

        def _basis0(g, _):
            _basis(0, 0, g)
            return 0
        lax.fori_loop(0, GPS, _basis0, 0)
        if nsc > 1:
            _issue_inputs(1)
        _gather_desc(0, 0, 0).start()

        def _superchunk(i, _):
            b = i % 2
            @pl.when(i + 1 < nsc)
            def _():
                _wait_inputs(i + 1)

            for gg in range(GPS):
                pltpu.async_copy(ones.at[pl.ds(0, CH)],
                                 deg_sh.at[rowv.at[b, gg]], dsem, add=True)

            def _pair(gp, _2):
                _group(i, b, gp, 2 * gp, 0)
                _group(i, b, gp, 2 * gp + 1, 1)
                return 0
            lax.fori_loop(0, GPS // 2, _pair, 0)

            _scat_desc(b, GPS - 2, 0).wait()
            _scat_desc(b, GPS - 1, 1).wait()
            for gg in range(GPS):
                pltpu.make_async_copy(ones.at[pl.ds(0, CH)],
                                      deg_sh.at[rowv.at[b, gg]], dsem).wait()

            @pl.when(i + 2 < nsc)
            def _():
                _issue_inputs(i + 2)
            return 0

        lax.fori_loop(0, nsc, _superchunk, 0)
        plsc.subcore_barrier()

        for off0, cl in chunks:
            off = sid * rows_per_sub + off0
            pltpu.sync_copy(accum_sh.at[pl.ds(off, cl)],
                            acc_out.at[cid, pl.ds(off, cl)])
        pltpu.sync_copy(
            deg_sh.at[pl.ds(sid * deg_per_sub, deg_per_sub)],
            deg_out.at[pl.ds(cid * NDEG + sid * deg_per_sub, deg_per_sub)])

    return sc_kernel


def _combine_body(a_ref, d_ref, r_ref, b_ref, o_ref):
    a = a_ref[0] + a_ref[1]
    d = d_ref[0] + d_ref[1]
    o_ref[...] = a / jnp.maximum(d, 1.0) + r_ref[...] + b_ref[...]


def _combine(acc, deg, root, bias):
    return pl.pallas_call(
        _combine_body,
        grid=(N // BN,),
        in_specs=[
            pl.BlockSpec((NC, BN, F), lambda i: (0, i, 0)),
            pl.BlockSpec((NC, BN, 1), lambda i: (0, i, 0)),
            pl.BlockSpec((BN, F), lambda i: (i, 0)),
            pl.BlockSpec((1, F), lambda i: (0, 0)),
        ],
        out_specs=pl.BlockSpec((BN, F), lambda i: (i, 0)),
        out_shape=jax.ShapeDtypeStruct((N, F), jnp.float32),
    )(acc, deg, root, bias)


def kernel(x, edge_index, pseudo, weight, bias):
    E = edge_index.shape[1]
    ept = -(-E // NW)
    nsc = -(-ept // SCB)
    ept_pad = nsc * SCB
    e_pad = ept_pad * NW

    xw = _compute_xw(x, weight)
    xw_flat = xw.reshape(KT * N, F)
    root = xw[K]

    row = edge_index[0]
    col = edge_index[1]
    pad = e_pad - E
    rowp = jnp.pad(row, (0, pad),
                   constant_values=NACC - 1).reshape(e_pad // CH, CH)
    colp = jnp.pad(col, (0, pad))
    pp = [jnp.pad(pseudo[:, d], (0, pad)) for d in range(DIM)]

    sc = _make_sc_kernel(E, ept_pad, nsc)
    acc, deg = sc(pp[0], pp[1], pp[2], rowp, colp, xw_flat)

    deg3 = deg.reshape(NC, NDEG)[:, :N].reshape(NC, N, 1)
    return _combine(acc, deg3, root, bias.reshape(1, F))


# speedup vs baseline: 1.7205x; 1.0891x over previous
import functools
import itertools

import jax
import jax.numpy as jnp
from jax import lax
from jax.experimental import pallas as pl
from jax.experimental.pallas import tpu as pltpu
from jax.experimental.pallas import tpu_sc as plsc

DIM = 3
KS = 4
K = KS ** DIM
KT = K + 1
F = 128
N = 10000
NC = 2
NS = 16
NW = NC * NS
CH = 16
S = 8
GPS = 16
SCB = GPS * CH
NDEG = 10240
NACC = 10112
BN = 400


def _xw_body(x_ref, w_ref, o_ref):
    xb = x_ref[...]
    for k in range(KT):
        o_ref[k] = jnp.dot(xb, w_ref[k], preferred_element_type=jnp.float32)


def _compute_xw(x, weight):
    return pl.pallas_call(
        _xw_body,
        grid=(N // BN,),
        in_specs=[
            pl.BlockSpec((BN, F), lambda nb: (nb, 0)),
            pl.BlockSpec((KT, F, F), lambda nb: (0, 0, 0)),
        ],
        out_specs=pl.BlockSpec((KT, BN, F), lambda nb: (0, nb, 0)),
        out_shape=jax.ShapeDtypeStruct((KT, N, F), jnp.float32),
    )(x.astype(jnp.bfloat16), weight.astype(jnp.bfloat16))


def _splat(vec, lane):
    idx = jnp.full((16, 1), lane, jnp.int32)
    dnums = lax.GatherDimensionNumbers(
        offset_dims=(), collapsed_slice_dims=(0,), start_index_map=(0,))
    return lax.gather(vec, idx, dnums, (1,),
                      mode=lax.GatherScatterMode.PROMISE_IN_BOUNDS)


def _make_sc_kernel(E, ept_pad, nsc):
    mesh = plsc.VectorSubcoreMesh(core_axis_name="c", subcore_axis_name="s")
    rows_per_sub = NACC // NS
    chunks = ((0, 128), (128, 128), (256, 128), (384, 128), (512, 120))
    deg_per_sub = NDEG // NS

    @functools.partial(
        pl.kernel,
        out_type=(
            jax.ShapeDtypeStruct((NC, NACC, F), jnp.float32),
            jax.ShapeDtypeStruct((NC * NDEG,), jnp.float32),
        ),
        mesh=mesh,
        scratch_types=[
            pltpu.VMEM_SHARED((NACC, F), jnp.float32),
            pltpu.VMEM_SHARED((NDEG,), jnp.float32),
            pltpu.VMEM((2 * SCB,), jnp.float32),
            pltpu.VMEM((2 * SCB,), jnp.float32),
            pltpu.VMEM((2 * SCB,), jnp.float32),
            pltpu.VMEM((2, GPS, CH), jnp.int32),
            pltpu.VMEM((2 * SCB,), jnp.int32),
            pltpu.VMEM((2 * SCB * S,), jnp.int32),
            pltpu.VMEM((2, DIM, SCB), jnp.float32),
            pltpu.VMEM((2, CH * S, F), jnp.float32),
            pltpu.VMEM((2, CH, F), jnp.float32),
            pltpu.VMEM((128,), jnp.float32),
            pltpu.VMEM((128,), jnp.float32),
            pltpu.SemaphoreType.DMA,
            pltpu.SemaphoreType.DMA,
            pltpu.SemaphoreType.DMA,
            pltpu.SemaphoreType.DMA,
            pltpu.SemaphoreType.DMA,
            pltpu.SemaphoreType.DMA,
        ],
    )
    def sc_kernel(p0_hbm, p1_hbm, p2_hbm, row_hbm, col_hbm, xw_hbm,
                  acc_out, deg_out,
                  accum_sh, deg_sh, p0v, p1v, p2v, rowv, colv, idxv,
                  frv, rows2, eout2, dz, ones,
                  isem, gsem0, gsem1, ssem0, ssem1, dsem):
        cid = lax.axis_index("c")
        sid = lax.axis_index("s")
        wid = sid * NC + cid
        gsems = (gsem0, gsem1)
        ssems = (ssem0, ssem1)

        def _zloop(i, _):
            for j in range(F // 16):
                eout2[0, i, pl.ds(16 * j, 16)] = jnp.zeros((16,), jnp.float32)
            return 0
        lax.fori_loop(0, CH, _zloop, 0)
        for j in range(128 // 16):
            dz[pl.ds(16 * j, 16)] = jnp.zeros((16,), jnp.float32)
            ones[pl.ds(16 * j, 16)] = jnp.ones((16,), jnp.float32)

        for k in range(rows_per_sub // CH):
            pltpu.sync_copy(
                eout2.at[0],
                accum_sh.at[pl.ds(sid * rows_per_sub + k * CH, CH)])
        pltpu.sync_copy(
            eout2.at[0, pl.ds(0, rows_per_sub % CH)],
            accum_sh.at[pl.ds(sid * rows_per_sub
                              + (rows_per_sub // CH) * CH,
                              rows_per_sub % CH)])
        for k in range(deg_per_sub // 128):
            pltpu.sync_copy(dz, deg_sh.at[pl.ds(sid * deg_per_sub + k * 128,
                                                128)])
        plsc.subcore_barrier()

        base0 = wid * ept_pad
        rbase0 = (wid * ept_pad) // CH
        lanes = lax.iota(jnp.int32, 16)
        inps = ((p0_hbm, p0v), (p1_hbm, p1v), (p2_hbm, p2v),
                (col_hbm, colv))

        def _issue_inputs(i):
            off = base0 + i * SCB
            nb = i % 2
            for hbm, buf in inps:
                pltpu.async_copy(hbm.at[pl.ds(off, SCB)],
                                 buf.at[pl.ds(nb * SCB, SCB)], isem)
            pltpu.async_copy(
                row_hbm.at[pl.ds(pl.multiple_of(rbase0 + i * GPS, 8), GPS), :],
                rowv.at[nb], isem)

        def _wait_inputs(i):
            off = base0 + i * SCB
            nb = i % 2
            for hbm, buf in inps:
                pltpu.make_async_copy(
                    hbm.at[pl.ds(off, SCB)],
                    buf.at[pl.ds(nb * SCB, SCB)], isem).wait()
            pltpu.make_async_copy(
                row_hbm.at[pl.ds(pl.multiple_of(rbase0 + i * GPS, 8), GPS), :],
                rowv.at[nb], isem).wait()

        def _basis(i, nb, g):
            e0 = g * CH
            colx = colv[pl.ds(nb * SCB + e0, CH)]
            lo = []
            for d, pv in enumerate((p0v, p1v, p2v)):
                v = pv[pl.ds(nb * SCB + e0, CH)] * float(KS - 1)
                li = jnp.minimum(v.astype(jnp.int32), KS - 2)
                lo.append(li)
                frv[nb, d, pl.ds(e0, CH)] = v - li.astype(jnp.float32)
            for sidx, bits in enumerate(itertools.product((0, 1), repeat=DIM)):
                idxl = jnp.zeros((16,), jnp.int32)
                for d, bit in enumerate(bits):
                    idxl = idxl + (lo[d] + bit) * (KS ** (DIM - 1 - d))
                idxv[pl.ds(nb * (SCB * S) + g * (CH * S) + sidx * CH,
                           CH)] = idxl * N + colx

        def _gather_desc(nb, g, p):
            return pltpu.make_async_copy(
                xw_hbm.at[idxv.at[pl.ds(nb * (SCB * S) + g * (CH * S), CH * S)]],
                rows2.at[p], gsems[p])

        def _scat_desc(b, g, p):
            return pltpu.make_async_copy(
                eout2.at[p], accum_sh.at[rowv.at[b, g]], ssems[p])

        def _compute(b, g, p):
            fv = [frv[b, d, pl.ds(g * CH, CH)] for d in range(DIM)]

            @plsc.parallel_loop(0, CH, step=1, unroll=1)
            def _edge(eh):
                for e2 in range(1):
                    e = eh + e2
                    f0 = _splat(fv[0], e)
                    f1 = _splat(fv[1], e)
                    f2 = _splat(fv[2], e)
                    g0, g1, g2 = 1.0 - f0, 1.0 - f1, 1.0 - f2
                    t00, t01 = g0 * g1, g0 * f1
                    t10, t11 = f0 * g1, f0 * f1
                    sp = [t00 * g2, t00 * f2, t01 * g2, t01 * f2,
                          t10 * g2, t10 * f2, t11 * g2, t11 * f2]
                    for f8 in range(F // 16):
                        acc = sp[0] * rows2[p, 0 * CH + e, pl.ds(f8 * 16, 16)]
                        for si in range(1, S):
                            acc = acc + sp[si] * rows2[p, si * CH + e,
                                                       pl.ds(f8 * 16, 16)]
                        eout2[p, e, pl.ds(f8 * 16, 16)] = acc

        def _group(i, b, gp, g, p):
            if p == 0:
                _gather_desc(b, g + 1, 1).start()
            else:
                @pl.when(gp < GPS // 2 - 1)
                def _():
                    _gather_desc(b, g + 1, 0).start()

                @pl.when(jnp.logical_and(gp == GPS // 2 - 1, i < nsc - 1))
                def _():
                    _gather_desc(1 - b, 0, 0).start()
            @pl.when(gp >= 1)
            def _():
                _scat_desc(b, g - 2, p).wait()
            _gather_desc(b, g, p).wait()
            _compute(b, g, p)
            pltpu.async_copy(eout2.at[p], accum_sh.at[rowv.at[b, g]],
                             ssems[p], add=True)
            _basis(i + 1, 1 - b, g)

        _issue_inputs(0)
        _wait_inputs(0)

        def _basis0(g, _):
            _basis(0, 0, g)
            return 0
        lax.fori_loop(0, GPS, _basis0, 0)
        if nsc > 1:
            _issue_inputs(1)
        _gather_desc(0, 0, 0).start()

        def _superchunk(i, _):
            b = i % 2
            @pl.when(i + 1 < nsc)
            def _():
                _wait_inputs(i + 1)

            for gg in range(GPS):
                pltpu.async_copy(ones.at[pl.ds(0, CH)],
                                 deg_sh.at[rowv.at[b, gg]], dsem, add=True)

            def _pair(gp, _2):
                _group(i, b, gp, 2 * gp, 0)
                _group(i, b, gp, 2 * gp + 1, 1)
                return 0
            lax.fori_loop(0, GPS // 2, _pair, 0)

            _scat_desc(b, GPS - 2, 0).wait()
            _scat_desc(b, GPS - 1, 1).wait()
            for gg in range(GPS):
                pltpu.make_async_copy(ones.at[pl.ds(0, CH)],
                                      deg_sh.at[rowv.at[b, gg]], dsem).wait()

            @pl.when(i + 2 < nsc)
            def _():
                _issue_inputs(i + 2)
            return 0

        lax.fori_loop(0, nsc, _superchunk, 0)
        plsc.subcore_barrier()

        for off0, cl in chunks:
            off = sid * rows_per_sub + off0
            pltpu.sync_copy(accum_sh.at[pl.ds(off, cl)],
                            acc_out.at[cid, pl.ds(off, cl)])
        pltpu.sync_copy(
            deg_sh.at[pl.ds(sid * deg_per_sub, deg_per_sub)],
            deg_out.at[pl.ds(cid * NDEG + sid * deg_per_sub, deg_per_sub)])

    return sc_kernel


def _combine_body(a_ref, d_ref, r_ref, b_ref, o_ref):
    a = a_ref[0] + a_ref[1]
    d = d_ref[0] + d_ref[1]
    o_ref[...] = a / jnp.maximum(d, 1.0) + r_ref[...] + b_ref[...]


def _combine(acc, deg, root, bias):
    return pl.pallas_call(
        _combine_body,
        grid=(N // BN,),
        in_specs=[
            pl.BlockSpec((NC, BN, F), lambda i: (0, i, 0)),
            pl.BlockSpec((NC, BN, 1), lambda i: (0, i, 0)),
            pl.BlockSpec((BN, F), lambda i: (i, 0)),
            pl.BlockSpec((1, F), lambda i: (0, 0)),
        ],
        out_specs=pl.BlockSpec((BN, F), lambda i: (i, 0)),
        out_shape=jax.ShapeDtypeStruct((N, F), jnp.float32),
    )(acc, deg, root, bias)


def kernel(x, edge_index, pseudo, weight, bias):
    E = edge_index.shape[1]
    ept = -(-E // NW)
    nsc = -(-ept // SCB)
    ept_pad = nsc * SCB
    e_pad = ept_pad * NW

    xw = _compute_xw(x, weight)
    xw_flat = xw.reshape(KT * N, F)
    root = xw[K]

    row = edge_index[0]
    col = edge_index[1]
    pad = e_pad - E
    rowp = jnp.pad(row, (0, pad),
                   constant_values=NACC - 1).reshape(e_pad // CH, CH)
    colp = jnp.pad(col, (0, pad))
    pp = [jnp.pad(pseudo[:, d], (0, pad)) for d in range(DIM)]

    sc = _make_sc_kernel(E, ept_pad, nsc)
    acc, deg = sc(pp[0], pp[1], pp[2], rowp, colp, xw_flat)

    deg3 = deg.reshape(NC, NDEG)[:, :N].reshape(NC, N, 1)
    return _combine(acc, deg3, root, bias.reshape(1, F))


# speedup vs baseline: 1.8307x; 1.0640x over previous
import functools
import itertools

import jax
import jax.numpy as jnp
from jax import lax
from jax.experimental import pallas as pl
from jax.experimental.pallas import tpu as pltpu
from jax.experimental.pallas import tpu_sc as plsc

DIM = 3
KS = 4
K = KS ** DIM
KT = K + 1
F = 128
N = 10000
NC = 2
NS = 16
NW = NC * NS
CH = 16
S = 8
GPS = 16
SCB = GPS * CH
NDEG = 10240
NACC = 10112
BN = 400


def _xw_body(x_ref, w_ref, o_ref):
    xb = x_ref[...]
    for k in range(KT):
        o_ref[k] = jnp.dot(xb, w_ref[k], preferred_element_type=jnp.float32)


def _compute_xw(x, weight):
    return pl.pallas_call(
        _xw_body,
        grid=(N // BN,),
        in_specs=[
            pl.BlockSpec((BN, F), lambda nb: (nb, 0)),
            pl.BlockSpec((KT, F, F), lambda nb: (0, 0, 0)),
        ],
        out_specs=pl.BlockSpec((KT, BN, F), lambda nb: (0, nb, 0)),
        out_shape=jax.ShapeDtypeStruct((KT, N, F), jnp.float32),
    )(x.astype(jnp.bfloat16), weight.astype(jnp.bfloat16))


def _splat(vec, lane):
    idx = jnp.full((16, 1), lane, jnp.int32)
    dnums = lax.GatherDimensionNumbers(
        offset_dims=(), collapsed_slice_dims=(0,), start_index_map=(0,))
    return lax.gather(vec, idx, dnums, (1,),
                      mode=lax.GatherScatterMode.PROMISE_IN_BOUNDS)


def _make_sc_kernel(E, nsc0, nsc1):
    mesh = plsc.VectorSubcoreMesh(core_axis_name="c", subcore_axis_name="s")
    rows_per_sub = NACC // NS
    chunks = ((0, 128), (128, 128), (256, 128), (384, 128), (512, 120))
    deg_per_sub = NDEG // NS

    @functools.partial(
        pl.kernel,
        out_type=(
            jax.ShapeDtypeStruct((NC, NACC, F), jnp.float32),
            jax.ShapeDtypeStruct((NC * NDEG,), jnp.float32),
        ),
        mesh=mesh,
        scratch_types=[
            pltpu.VMEM_SHARED((NACC, F), jnp.float32),
            pltpu.VMEM_SHARED((NDEG,), jnp.float32),
            pltpu.VMEM((2 * SCB,), jnp.float32),
            pltpu.VMEM((2 * SCB,), jnp.float32),
            pltpu.VMEM((2 * SCB,), jnp.float32),
            pltpu.VMEM((2, GPS, CH), jnp.int32),
            pltpu.VMEM((2 * SCB,), jnp.int32),
            pltpu.VMEM((2 * SCB * S,), jnp.int32),
            pltpu.VMEM((2, DIM, SCB), jnp.float32),
            pltpu.VMEM((2, CH * S, F), jnp.float32),
            pltpu.VMEM((2, CH, F), jnp.float32),
            pltpu.VMEM((128,), jnp.float32),
            pltpu.VMEM((128,), jnp.float32),
            pltpu.SemaphoreType.DMA,
            pltpu.SemaphoreType.DMA,
            pltpu.SemaphoreType.DMA,
            pltpu.SemaphoreType.DMA,
            pltpu.SemaphoreType.DMA,
            pltpu.SemaphoreType.DMA,
        ],
    )
    def sc_kernel(p0_hbm, p1_hbm, p2_hbm, row_hbm, col_hbm, xw_hbm,
                  acc_out, deg_out,
                  accum_sh, deg_sh, p0v, p1v, p2v, rowv, colv, idxv,
                  frv, rows2, eout2, dz, ones,
                  isem, gsem0, gsem1, ssem0, ssem1, dsem):
        cid = lax.axis_index("c")
        sid = lax.axis_index("s")
        gsems = (gsem0, gsem1)
        ssems = (ssem0, ssem1)
        nsc = jnp.where(cid == 0, nsc0, nsc1)
        base0 = jnp.where(cid == 0, sid * (nsc0 * SCB),
                          NS * (nsc0 * SCB) + sid * (nsc1 * SCB))

        def _zloop(i, _):
            for j in range(F // 16):
                eout2[0, i, pl.ds(16 * j, 16)] = jnp.zeros((16,), jnp.float32)
            return 0
        lax.fori_loop(0, CH, _zloop, 0)
        for j in range(128 // 16):
            dz[pl.ds(16 * j, 16)] = jnp.zeros((16,), jnp.float32)
            ones[pl.ds(16 * j, 16)] = jnp.ones((16,), jnp.float32)

        for k in range(rows_per_sub // CH):
            pltpu.sync_copy(
                eout2.at[0],
                accum_sh.at[pl.ds(sid * rows_per_sub + k * CH, CH)])
        pltpu.sync_copy(
            eout2.at[0, pl.ds(0, rows_per_sub % CH)],
            accum_sh.at[pl.ds(sid * rows_per_sub
                              + (rows_per_sub // CH) * CH,
                              rows_per_sub % CH)])
        for k in range(deg_per_sub // 128):
            pltpu.sync_copy(dz, deg_sh.at[pl.ds(sid * deg_per_sub + k * 128,
                                                128)])
        plsc.subcore_barrier()

        rbase0 = base0 // CH
        lanes = lax.iota(jnp.int32, 16)
        inps = ((p0_hbm, p0v), (p1_hbm, p1v), (p2_hbm, p2v),
                (col_hbm, colv))

        def _issue_inputs(i):
            off = base0 + i * SCB
            nb = i % 2
            for hbm, buf in inps:
                pltpu.async_copy(hbm.at[pl.ds(off, SCB)],
                                 buf.at[pl.ds(nb * SCB, SCB)], isem)
            pltpu.async_copy(
                row_hbm.at[pl.ds(pl.multiple_of(rbase0 + i * GPS, 8), GPS), :],
                rowv.at[nb], isem)

        def _wait_inputs(i):
            off = base0 + i * SCB
            nb = i % 2
            for hbm, buf in inps:
                pltpu.make_async_copy(
                    hbm.at[pl.ds(off, SCB)],
                    buf.at[pl.ds(nb * SCB, SCB)], isem).wait()
            pltpu.make_async_copy(
                row_hbm.at[pl.ds(pl.multiple_of(rbase0 + i * GPS, 8), GPS), :],
                rowv.at[nb], isem).wait()

        def _basis(i, nb, g):
            e0 = g * CH
            colx = colv[pl.ds(nb * SCB + e0, CH)]
            lo = []
            for d, pv in enumerate((p0v, p1v, p2v)):
                v = pv[pl.ds(nb * SCB + e0, CH)] * float(KS - 1)
                li = jnp.minimum(v.astype(jnp.int32), KS - 2)
                lo.append(li)
                frv[nb, d, pl.ds(e0, CH)] = v - li.astype(jnp.float32)
            for sidx, bits in enumerate(itertools.product((0, 1), repeat=DIM)):
                idxl = jnp.zeros((16,), jnp.int32)
                for d, bit in enumerate(bits):
                    idxl = idxl + (lo[d] + bit) * (KS ** (DIM - 1 - d))
                idxv[pl.ds(nb * (SCB * S) + g * (CH * S) + sidx * CH,
                           CH)] = idxl * N + colx

        def _gather_desc(nb, g, p):
            return pltpu.make_async_copy(
                xw_hbm.at[idxv.at[pl.ds(nb * (SCB * S) + g * (CH * S), CH * S)]],
                rows2.at[p], gsems[p])

        def _scat_desc(b, g, p):
            return pltpu.make_async_copy(
                eout2.at[p], accum_sh.at[rowv.at[b, g]], ssems[p])

        def _compute(b, g, p):
            fv = [frv[b, d, pl.ds(g * CH, CH)] for d in range(DIM)]

            @plsc.parallel_loop(0, CH, step=1, unroll=1)
            def _edge(eh):
                for e2 in range(1):
                    e = eh + e2
                    f0 = _splat(fv[0], e)
                    f1 = _splat(fv[1], e)
                    f2 = _splat(fv[2], e)
                    g0, g1, g2 = 1.0 - f0, 1.0 - f1, 1.0 - f2
                    t00, t01 = g0 * g1, g0 * f1
                    t10, t11 = f0 * g1, f0 * f1
                    sp = [t00 * g2, t00 * f2, t01 * g2, t01 * f2,
                          t10 * g2, t10 * f2, t11 * g2, t11 * f2]
                    for f8 in range(F // 16):
                        acc = sp[0] * rows2[p, 0 * CH + e, pl.ds(f8 * 16, 16)]
                        for si in range(1, S):
                            acc = acc + sp[si] * rows2[p, si * CH + e,
                                                       pl.ds(f8 * 16, 16)]
                        eout2[p, e, pl.ds(f8 * 16, 16)] = acc

        def _group(i, b, gp, g, p):
            if p == 0:
                _gather_desc(b, g + 1, 1).start()
            else:
                @pl.when(gp < GPS // 2 - 1)
                def _():
                    _gather_desc(b, g + 1, 0).start()

                @pl.when(jnp.logical_and(gp == GPS // 2 - 1, i < nsc - 1))
                def _():
                    _gather_desc(1 - b, 0, 0).start()
            @pl.when(gp >= 1)
            def _():
                _scat_desc(b, g - 2, p).wait()
            _gather_desc(b, g, p).wait()
            _compute(b, g, p)
            pltpu.async_copy(eout2.at[p], accum_sh.at[rowv.at[b, g]],
                             ssems[p], add=True)
            _basis(i + 1, 1 - b, g)

        _issue_inputs(0)
        _wait_inputs(0)

        def _basis0(g, _):
            _basis(0, 0, g)
            return 0
        lax.fori_loop(0, GPS, _basis0, 0)
        _issue_inputs(1)
        _gather_desc(0, 0, 0).start()

        def _superchunk(i, _):
            b = i % 2
            @pl.when(i + 1 < nsc)
            def _():
                _wait_inputs(i + 1)

            for gg in range(GPS):
                pltpu.async_copy(ones.at[pl.ds(0, CH)],
                                 deg_sh.at[rowv.at[b, gg]], dsem, add=True)

            def _pair(gp, _2):
                _group(i, b, gp, 2 * gp, 0)
                _group(i, b, gp, 2 * gp + 1, 1)
                return 0
            lax.fori_loop(0, GPS // 2, _pair, 0)

            _scat_desc(b, GPS - 2, 0).wait()
            _scat_desc(b, GPS - 1, 1).wait()
            for gg in range(GPS):
                pltpu.make_async_copy(ones.at[pl.ds(0, CH)],
                                      deg_sh.at[rowv.at[b, gg]], dsem).wait()

            @pl.when(i + 2 < nsc)
            def _():
                _issue_inputs(i + 2)
            return 0

        lax.fori_loop(0, nsc, _superchunk, 0)
        plsc.subcore_barrier()

        for off0, cl in chunks:
            off = sid * rows_per_sub + off0
            pltpu.sync_copy(accum_sh.at[pl.ds(off, cl)],
                            acc_out.at[cid, pl.ds(off, cl)])
        pltpu.sync_copy(
            deg_sh.at[pl.ds(sid * deg_per_sub, deg_per_sub)],
            deg_out.at[pl.ds(cid * NDEG + sid * deg_per_sub, deg_per_sub)])

    return sc_kernel


def _combine_body(a_ref, d_ref, r_ref, b_ref, o_ref):
    a = a_ref[0] + a_ref[1]
    d = d_ref[0] + d_ref[1]
    o_ref[...] = a / jnp.maximum(d, 1.0) + r_ref[...] + b_ref[...]


def _combine(acc, deg, root, bias):
    return pl.pallas_call(
        _combine_body,
        grid=(N // BN,),
        in_specs=[
            pl.BlockSpec((NC, BN, F), lambda i: (0, i, 0)),
            pl.BlockSpec((NC, BN, 1), lambda i: (0, i, 0)),
            pl.BlockSpec((BN, F), lambda i: (i, 0)),
            pl.BlockSpec((1, F), lambda i: (0, 0)),
        ],
        out_specs=pl.BlockSpec((BN, F), lambda i: (i, 0)),
        out_shape=jax.ShapeDtypeStruct((N, F), jnp.float32),
    )(acc, deg, root, bias)


def kernel(x, edge_index, pseudo, weight, bias):
    E = edge_index.shape[1]
    percol = NS * SCB
    nsc0 = max(2, round(0.6 * E / percol))
    nsc1 = max(2, -(-(E - nsc0 * percol) // percol))
    e_pad = (nsc0 + nsc1) * percol

    xw = _compute_xw(x, weight)
    xw_flat = xw.reshape(KT * N, F)
    root = xw[K]

    row = edge_index[0]
    col = edge_index[1]
    pad = e_pad - E
    rowp = jnp.pad(row, (0, pad),
                   constant_values=NACC - 1).reshape(e_pad // CH, CH)
    colp = jnp.pad(col, (0, pad))
    pp = [jnp.pad(pseudo[:, d], (0, pad)) for d in range(DIM)]

    sc = _make_sc_kernel(E, nsc0, nsc1)
    acc, deg = sc(pp[0], pp[1], pp[2], rowp, colp, xw_flat)

    deg3 = deg.reshape(NC, NDEG)[:, :N].reshape(NC, N, 1)
    return _combine(acc, deg3, root, bias.reshape(1, F))


# speedup vs baseline: 1.8624x; 1.0174x over previous
import functools
import itertools

import jax
import jax.numpy as jnp
from jax import lax
from jax.experimental import pallas as pl
from jax.experimental.pallas import tpu as pltpu
from jax.experimental.pallas import tpu_sc as plsc

DIM = 3
KS = 4
K = KS ** DIM
KT = K + 1
F = 128
N = 10000
NC = 2
NS = 16
NW = NC * NS
CH = 16
S = 8
GPS = 16
SCB = GPS * CH
NDEG = 10240
NACC = 10112
BN = 400


def _xw_body(x_ref, w_ref, o_ref):
    xb = x_ref[...]
    for k in range(KT):
        o_ref[k] = jnp.dot(xb, w_ref[k], preferred_element_type=jnp.float32)


def _compute_xw(x, weight):
    return pl.pallas_call(
        _xw_body,
        grid=(N // BN,),
        in_specs=[
            pl.BlockSpec((BN, F), lambda nb: (nb, 0)),
            pl.BlockSpec((KT, F, F), lambda nb: (0, 0, 0)),
        ],
        out_specs=pl.BlockSpec((KT, BN, F), lambda nb: (0, nb, 0)),
        out_shape=jax.ShapeDtypeStruct((KT, N, F), jnp.float32),
    )(x.astype(jnp.bfloat16), weight.astype(jnp.bfloat16))


def _splat(vec, lane):
    idx = jnp.full((16, 1), lane, jnp.int32)
    dnums = lax.GatherDimensionNumbers(
        offset_dims=(), collapsed_slice_dims=(0,), start_index_map=(0,))
    return lax.gather(vec, idx, dnums, (1,),
                      mode=lax.GatherScatterMode.PROMISE_IN_BOUNDS)


def _make_sc_kernel(E, nsc0, nsc1):
    mesh = plsc.VectorSubcoreMesh(core_axis_name="c", subcore_axis_name="s")
    rows_per_sub = NACC // NS
    chunks = ((0, 128), (128, 128), (256, 128), (384, 128), (512, 120))
    deg_per_sub = NDEG // NS

    @functools.partial(
        pl.kernel,
        out_type=(
            jax.ShapeDtypeStruct((NC, NACC, F), jnp.float32),
            jax.ShapeDtypeStruct((NC * NDEG,), jnp.float32),
        ),
        mesh=mesh,
        scratch_types=[
            pltpu.VMEM_SHARED((NACC, F), jnp.float32),
            pltpu.VMEM_SHARED((NDEG,), jnp.float32),
            pltpu.VMEM((2 * SCB,), jnp.float32),
            pltpu.VMEM((2 * SCB,), jnp.float32),
            pltpu.VMEM((2 * SCB,), jnp.float32),
            pltpu.VMEM((2, GPS, CH), jnp.int32),
            pltpu.VMEM((2 * SCB,), jnp.int32),
            pltpu.VMEM((2 * SCB * S,), jnp.int32),
            pltpu.VMEM((2, DIM, SCB), jnp.float32),
            pltpu.VMEM((2, CH * S, F), jnp.float32),
            pltpu.VMEM((2, CH, F), jnp.float32),
            pltpu.VMEM((128,), jnp.float32),
            pltpu.VMEM((128,), jnp.float32),
            pltpu.SemaphoreType.DMA,
            pltpu.SemaphoreType.DMA,
            pltpu.SemaphoreType.DMA,
            pltpu.SemaphoreType.DMA,
            pltpu.SemaphoreType.DMA,
            pltpu.SemaphoreType.DMA,
        ],
    )
    def sc_kernel(p0_hbm, p1_hbm, p2_hbm, row_hbm, col_hbm, xw_hbm,
                  acc_out, deg_out,
                  accum_sh, deg_sh, p0v, p1v, p2v, rowv, colv, idxv,
                  frv, rows2, eout2, dz, ones,
                  isem, gsem0, gsem1, ssem0, ssem1, dsem):
        cid = lax.axis_index("c")
        sid = lax.axis_index("s")
        gsems = (gsem0, gsem1)
        ssems = (ssem0, ssem1)
        nsc = jnp.where(cid == 0, nsc0, nsc1)
        base0 = jnp.where(cid == 0, sid * (nsc0 * SCB),
                          NS * (nsc0 * SCB) + sid * (nsc1 * SCB))

        def _zloop(i, _):
            for j in range(F // 16):
                eout2[0, i, pl.ds(16 * j, 16)] = jnp.zeros((16,), jnp.float32)
            return 0
        lax.fori_loop(0, CH, _zloop, 0)
        for j in range(128 // 16):
            dz[pl.ds(16 * j, 16)] = jnp.zeros((16,), jnp.float32)
            ones[pl.ds(16 * j, 16)] = jnp.ones((16,), jnp.float32)

        for k in range(rows_per_sub // CH):
            pltpu.sync_copy(
                eout2.at[0],
                accum_sh.at[pl.ds(sid * rows_per_sub + k * CH, CH)])
        pltpu.sync_copy(
            eout2.at[0, pl.ds(0, rows_per_sub % CH)],
            accum_sh.at[pl.ds(sid * rows_per_sub
                              + (rows_per_sub // CH) * CH,
                              rows_per_sub % CH)])
        for k in range(deg_per_sub // 128):
            pltpu.sync_copy(dz, deg_sh.at[pl.ds(sid * deg_per_sub + k * 128,
                                                128)])
        plsc.subcore_barrier()

        rbase0 = base0 // CH
        lanes = lax.iota(jnp.int32, 16)
        inps = ((p0_hbm, p0v), (p1_hbm, p1v), (p2_hbm, p2v),
                (col_hbm, colv))

        def _issue_inputs(i):
            off = base0 + i * SCB
            nb = i % 2
            for hbm, buf in inps:
                pltpu.async_copy(hbm.at[pl.ds(off, SCB)],
                                 buf.at[pl.ds(nb * SCB, SCB)], isem)
            pltpu.async_copy(
                row_hbm.at[pl.ds(pl.multiple_of(rbase0 + i * GPS, 8), GPS), :],
                rowv.at[nb], isem)

        def _wait_inputs(i):
            off = base0 + i * SCB
            nb = i % 2
            for hbm, buf in inps:
                pltpu.make_async_copy(
                    hbm.at[pl.ds(off, SCB)],
                    buf.at[pl.ds(nb * SCB, SCB)], isem).wait()
            pltpu.make_async_copy(
                row_hbm.at[pl.ds(pl.multiple_of(rbase0 + i * GPS, 8), GPS), :],
                rowv.at[nb], isem).wait()

        def _basis(i, nb, g):
            e0 = g * CH
            colx = colv[pl.ds(nb * SCB + e0, CH)]
            lo = []
            for d, pv in enumerate((p0v, p1v, p2v)):
                v = pv[pl.ds(nb * SCB + e0, CH)] * float(KS - 1)
                li = jnp.minimum(v.astype(jnp.int32), KS - 2)
                lo.append(li)
                frv[nb, d, pl.ds(e0, CH)] = v - li.astype(jnp.float32)
            for sidx, bits in enumerate(itertools.product((0, 1), repeat=DIM)):
                idxl = jnp.zeros((16,), jnp.int32)
                for d, bit in enumerate(bits):
                    idxl = idxl + (lo[d] + bit) * (KS ** (DIM - 1 - d))
                idxv[pl.ds(nb * (SCB * S) + g * (CH * S) + sidx * CH,
                           CH)] = idxl * N + colx

        def _gather_desc(nb, g, p):
            return pltpu.make_async_copy(
                xw_hbm.at[idxv.at[pl.ds(nb * (SCB * S) + g * (CH * S), CH * S)]],
                rows2.at[p], gsems[p])

        def _scat_desc(b, g, p):
            return pltpu.make_async_copy(
                eout2.at[p], accum_sh.at[rowv.at[b, g]], ssems[p])

        def _compute(b, g, p):
            fv = [frv[b, d, pl.ds(g * CH, CH)] for d in range(DIM)]

            @plsc.parallel_loop(0, CH, step=1, unroll=1)
            def _edge(eh):
                for e2 in range(1):
                    e = eh + e2
                    f0 = _splat(fv[0], e)
                    f1 = _splat(fv[1], e)
                    f2 = _splat(fv[2], e)
                    g0, g1, g2 = 1.0 - f0, 1.0 - f1, 1.0 - f2
                    t00, t01 = g0 * g1, g0 * f1
                    t10, t11 = f0 * g1, f0 * f1
                    sp = [t00 * g2, t00 * f2, t01 * g2, t01 * f2,
                          t10 * g2, t10 * f2, t11 * g2, t11 * f2]
                    for f8 in range(F // 16):
                        acc = sp[0] * rows2[p, 0 * CH + e, pl.ds(f8 * 16, 16)]
                        for si in range(1, S):
                            acc = acc + sp[si] * rows2[p, si * CH + e,
                                                       pl.ds(f8 * 16, 16)]
                        eout2[p, e, pl.ds(f8 * 16, 16)] = acc

        def _group(i, b, gp, g, p):
            if p == 0:
                _gather_desc(b, g + 1, 1).start()
            else:
                @pl.when(gp < GPS // 2 - 1)
                def _():
                    _gather_desc(b, g + 1, 0).start()

                @pl.when(jnp.logical_and(gp == GPS // 2 - 1, i < nsc - 1))
                def _():
                    _gather_desc(1 - b, 0, 0).start()
            @pl.when(gp >= 1)
            def _():
                _scat_desc(b, g - 2, p).wait()
            _gather_desc(b, g, p).wait()
            _compute(b, g, p)
            pltpu.async_copy(eout2.at[p], accum_sh.at[rowv.at[b, g]],
                             ssems[p], add=True)
            _basis(i + 1, 1 - b, g)

        _issue_inputs(0)
        _wait_inputs(0)

        def _basis0(g, _):
            _basis(0, 0, g)
            return 0
        lax.fori_loop(0, GPS, _basis0, 0)
        _issue_inputs(1)
        _gather_desc(0, 0, 0).start()

        def _superchunk(i, _):
            b = i % 2
            @pl.when(i + 1 < nsc)
            def _():
                _wait_inputs(i + 1)

            for gg in range(GPS):
                pltpu.async_copy(ones.at[pl.ds(0, CH)],
                                 deg_sh.at[rowv.at[b, gg]], dsem, add=True)

            def _pair(gp, _2):
                _group(i, b, gp, 2 * gp, 0)
                _group(i, b, gp, 2 * gp + 1, 1)
                return 0
            lax.fori_loop(0, GPS // 2, _pair, 0)

            _scat_desc(b, GPS - 2, 0).wait()
            _scat_desc(b, GPS - 1, 1).wait()
            for gg in range(GPS):
                pltpu.make_async_copy(ones.at[pl.ds(0, CH)],
                                      deg_sh.at[rowv.at[b, gg]], dsem).wait()

            @pl.when(i + 2 < nsc)
            def _():
                _issue_inputs(i + 2)
            return 0

        lax.fori_loop(0, nsc, _superchunk, 0)
        plsc.subcore_barrier()

        for off0, cl in chunks:
            off = sid * rows_per_sub + off0
            pltpu.sync_copy(accum_sh.at[pl.ds(off, cl)],
                            acc_out.at[cid, pl.ds(off, cl)])
        pltpu.sync_copy(
            deg_sh.at[pl.ds(sid * deg_per_sub, deg_per_sub)],
            deg_out.at[pl.ds(cid * NDEG + sid * deg_per_sub, deg_per_sub)])

    return sc_kernel


def _combine_body(a_ref, d_ref, r_ref, b_ref, o_ref):
    a = a_ref[0] + a_ref[1]
    d = d_ref[0] + d_ref[1]
    o_ref[...] = a / jnp.maximum(d, 1.0) + r_ref[...] + b_ref[...]


def _combine(acc, deg, root, bias):
    return pl.pallas_call(
        _combine_body,
        grid=(N // BN,),
        in_specs=[
            pl.BlockSpec((NC, BN, F), lambda i: (0, i, 0)),
            pl.BlockSpec((NC, BN, 1), lambda i: (0, i, 0)),
            pl.BlockSpec((BN, F), lambda i: (i, 0)),
            pl.BlockSpec((1, F), lambda i: (0, 0)),
        ],
        out_specs=pl.BlockSpec((BN, F), lambda i: (i, 0)),
        out_shape=jax.ShapeDtypeStruct((N, F), jnp.float32),
    )(acc, deg, root, bias)


def kernel(x, edge_index, pseudo, weight, bias):
    E = edge_index.shape[1]
    percol = NS * SCB
    nsc0 = max(2, round(0.61 * E / percol))
    nsc1 = max(2, -(-(E - nsc0 * percol) // percol))
    e_pad = (nsc0 + nsc1) * percol

    xw = _compute_xw(x, weight)
    xw_flat = xw.reshape(KT * N, F)
    root = xw[K]

    row = edge_index[0]
    col = edge_index[1]
    pad = e_pad - E
    rowp = jnp.pad(row, (0, pad),
                   constant_values=NACC - 1).reshape(e_pad // CH, CH)
    colp = jnp.pad(col, (0, pad))
    pp = [jnp.pad(pseudo[:, d], (0, pad)) for d in range(DIM)]

    sc = _make_sc_kernel(E, nsc0, nsc1)
    acc, deg = sc(pp[0], pp[1], pp[2], rowp, colp, xw_flat)

    deg3 = deg.reshape(NC, NDEG)[:, :N].reshape(NC, N, 1)
    return _combine(acc, deg3, root, bias.reshape(1, F))
